# baseline (speedup 1.0000x reference)
import functools

import jax
import jax.numpy as jnp
from jax import lax
from jax.experimental import pallas as pl
from jax.experimental.pallas import tpu as pltpu
from jax.experimental.pallas import tpu_sc as plsc

_BATCH = 16384
_NODES = 1024

_INFO = plsc.get_sparse_core_info()
_NC, _NS = _INFO.num_cores, _INFO.num_subcores
_NW = _NC * _NS
_ROWS_PER_W = _BATCH // _NW
_CH = 32
_NCHUNK = _ROWS_PER_W // _CH
_CHW = _CH * _NODES


def _vgather(v, idx):
    return lax.gather(
        v,
        idx[:, None],
        dimension_numbers=lax.GatherDimensionNumbers(
            offset_dims=(), collapsed_slice_dims=(0,), start_index_map=(0,)),
        slice_sizes=(1,),
        mode=lax.GatherScatterMode.PROMISE_IN_BOUNDS,
    )


def _tree_fix(buf, off, p1, p2, p4, p_hi, i16):
    lo = buf[pl.ds(off, 16)]
    lo = jnp.minimum(lo, _vgather(lo, p1))
    lo = jnp.minimum(lo, _vgather(lo, p2))
    lo = jnp.minimum(lo, _vgather(lo, p4))
    buf[pl.ds(off, 16)] = lo
    hi = buf[pl.ds(off + 16, 16)]
    gh = _vgather(lo, p_hi)
    hi = jnp.where(i16 < 15, jnp.minimum(hi, gh), hi)
    buf[pl.ds(off + 16, 16)] = hi


def _compute_chunk(buf, p1, p2, p4, p_hi, i16):
    def fix_row(r, c):
        _tree_fix(buf, r * _NODES, p1, p2, p4, p_hi, i16)
        return c

    lax.fori_loop(0, _CH, fix_row, 0, unroll=2)

    def sig(i, c):
        base = i * 64
        for j in range(4):
            off = base + j * 16
            v = buf[pl.ds(off, 16)]
            buf[pl.ds(off, 16)] = 1.0 / (1.0 + jnp.exp(-v))
        return c

    lax.fori_loop(0, _CHW // 64, sig, 0)


def _sc_body(x_hbm, o_hbm, b0, b1, ls0, ls1, ss0, ss1):
    wid = lax.axis_index("s") * _NC + lax.axis_index("c")
    w0 = wid * _ROWS_PER_W * _NODES

    i16 = lax.iota(jnp.int32, 16)
    par = lambda v: jnp.maximum((v - 1) >> 1, 0)
    p1 = par(i16)
    p2 = par(p1)
    p4 = par(par(p2))
    p_hi = (i16 + 15) >> 1

    bufs = (b0, b1)
    lsems = (ls0, ls1)
    ssems = (ss0, ss1)

    def ld(k):
        return pltpu.make_async_copy(
            x_hbm.at[pl.ds(w0 + k * _CHW, _CHW)], bufs[k % 2], lsems[k % 2])

    def st(k):
        return pltpu.make_async_copy(
            bufs[k % 2], o_hbm.at[pl.ds(w0 + k * _CHW, _CHW)], ssems[k % 2])

    ld(0).start()
    for k in range(_NCHUNK):
        if k + 1 < _NCHUNK:
            if k - 1 >= 0:
                st(k - 1).wait()
            ld(k + 1).start()
        ld(k).wait()
        _compute_chunk(bufs[k % 2], p1, p2, p4, p_hi, i16)
        st(k).start()
    st(_NCHUNK - 2).wait()
    st(_NCHUNK - 1).wait()


@jax.jit
def kernel(x):
    flat = x.reshape(-1)
    mesh = plsc.VectorSubcoreMesh(core_axis_name="c", subcore_axis_name="s")
    run = functools.partial(
        pl.kernel,
        mesh=mesh,
        out_type=jax.ShapeDtypeStruct((_BATCH * _NODES,), jnp.float32),
        scratch_types=[
            pltpu.VMEM((_CHW,), jnp.float32),
            pltpu.VMEM((_CHW,), jnp.float32),
            pltpu.SemaphoreType.DMA,
            pltpu.SemaphoreType.DMA,
            pltpu.SemaphoreType.DMA,
            pltpu.SemaphoreType.DMA,
        ],
    )(_sc_body)
    return run(flat).reshape(_BATCH, _NODES)


# speedup vs baseline: 1.5017x; 1.5017x over previous
import functools

import jax
import jax.numpy as jnp
from jax import lax
from jax.experimental import pallas as pl
from jax.experimental.pallas import tpu as pltpu
from jax.experimental.pallas import tpu_sc as plsc

_BATCH = 16384
_NODES = 1024

_INFO = plsc.get_sparse_core_info()
_NC, _NS = _INFO.num_cores, _INFO.num_subcores
_NW = _NC * _NS
_ROWS_PER_W = _BATCH // _NW
_CH = 32
_NCHUNK = _ROWS_PER_W // _CH
_CHW = _CH * _NODES


def _vgather(v, idx):
    return lax.gather(
        v,
        idx[:, None],
        dimension_numbers=lax.GatherDimensionNumbers(
            offset_dims=(), collapsed_slice_dims=(0,), start_index_map=(0,)),
        slice_sizes=(1,),
        mode=lax.GatherScatterMode.PROMISE_IN_BOUNDS,
    )


def _tree_fix(buf, off, p1, p2, p4, p_hi, i16):
    lo = buf[pl.ds(off, 16)]
    lo = jnp.minimum(lo, _vgather(lo, p1))
    lo = jnp.minimum(lo, _vgather(lo, p2))
    lo = jnp.minimum(lo, _vgather(lo, p4))
    buf[pl.ds(off, 16)] = lo
    hi = buf[pl.ds(off + 16, 16)]
    gh = _vgather(lo, p_hi)
    hi = jnp.where(i16 < 15, jnp.minimum(hi, gh), hi)
    buf[pl.ds(off + 16, 16)] = hi


def _compute_chunk(buf, p1, p2, p4, p_hi, i16):
    def fix_row(r, c):
        _tree_fix(buf, r * _NODES, p1, p2, p4, p_hi, i16)
        return c

    lax.fori_loop(0, _CH, fix_row, 0, unroll=2)

    def sig(i, c):
        base = i * 128
        vals = [buf[pl.ds(base + j * 16, 16)] for j in range(8)]
        outs = [1.0 / (1.0 + jnp.exp(-v)) for v in vals]
        for j in range(8):
            buf[pl.ds(base + j * 16, 16)] = outs[j]
        return c

    lax.fori_loop(0, _CHW // 128, sig, 0, unroll=4)


def _sc_body(x_hbm, o_hbm, b0, b1, ls0, ls1, ss0, ss1):
    wid = lax.axis_index("s") * _NC + lax.axis_index("c")
    w0 = wid * _ROWS_PER_W * _NODES

    i16 = lax.iota(jnp.int32, 16)
    par = lambda v: jnp.maximum((v - 1) >> 1, 0)
    p1 = par(i16)
    p2 = par(p1)
    p4 = par(par(p2))
    p_hi = (i16 + 15) >> 1

    bufs = (b0, b1)
    lsems = (ls0, ls1)
    ssems = (ss0, ss1)

    def ld(k):
        return pltpu.make_async_copy(
            x_hbm.at[pl.ds(w0 + k * _CHW, _CHW)], bufs[k % 2], lsems[k % 2])

    def st(k):
        return pltpu.make_async_copy(
            bufs[k % 2], o_hbm.at[pl.ds(w0 + k * _CHW, _CHW)], ssems[k % 2])

    ld(0).start()
    for k in range(_NCHUNK):
        if k + 1 < _NCHUNK:
            if k - 1 >= 0:
                st(k - 1).wait()
            ld(k + 1).start()
        ld(k).wait()
        _compute_chunk(bufs[k % 2], p1, p2, p4, p_hi, i16)
        st(k).start()
    st(_NCHUNK - 2).wait()
    st(_NCHUNK - 1).wait()


@jax.jit
def kernel(x):
    flat = x.reshape(-1)
    mesh = plsc.VectorSubcoreMesh(core_axis_name="c", subcore_axis_name="s")
    run = functools.partial(
        pl.kernel,
        mesh=mesh,
        out_type=jax.ShapeDtypeStruct((_BATCH * _NODES,), jnp.float32),
        scratch_types=[
            pltpu.VMEM((_CHW,), jnp.float32),
            pltpu.VMEM((_CHW,), jnp.float32),
            pltpu.SemaphoreType.DMA,
            pltpu.SemaphoreType.DMA,
            pltpu.SemaphoreType.DMA,
            pltpu.SemaphoreType.DMA,
        ],
    )(_sc_body)
    return run(flat).reshape(_BATCH, _NODES)
